# Initial kernel scaffold; baseline (speedup 1.0000x reference)
import jax
import jax.numpy as jnp
from jax.experimental import pallas as pl


def kernel(rel_selected, target_indices, source_indices, force, viscosity, W1, b1, W2, b2, median, contact_distance):
    raise NotImplementedError("write your pallas kernel here")


# speedup vs baseline: 30.3268x; 30.3268x over previous
import functools

import jax
import jax.numpy as jnp
from jax import lax
from jax.experimental import pallas as pl
from jax.experimental.pallas import tpu as pltpu
from jax.experimental.pallas import tpu_sc as plsc

N_NODES = 100000
N_EDGES = 3200000
NC = 2
NS = 16
NW = NC * NS
LANES = 16

PW = N_EDGES // NW
CB = 2000
NCHUNK = PW // CB
NP = 100096
NPT = NP // NS

BE = 25600
DPAD = 4


def _gather_body(fx_hbm, fy_hbm, fz_hbm, ti_hbm, si_hbm,
                 ftx_o, fty_o, ftz_o, fsx_o, fsy_o, fsz_o,
                 ti_v, si_v, c0_v, c1_v, c2_v, c3_v, c4_v, c5_v,
                 sem):
    cid = lax.axis_index("c")
    sid = lax.axis_index("s")
    wid = sid * NC + cid
    outs = (c0_v, c1_v, c2_v, c3_v, c4_v, c5_v)

    def chunk(c, _):
        base = wid * PW + c * CB
        pltpu.sync_copy(ti_hbm.at[pl.ds(base, CB)], ti_v)
        pltpu.sync_copy(si_hbm.at[pl.ds(base, CB)], si_v)
        cps = [
            pltpu.async_copy(fx_hbm.at[ti_v], c0_v, sem),
            pltpu.async_copy(fy_hbm.at[ti_v], c1_v, sem),
            pltpu.async_copy(fz_hbm.at[ti_v], c2_v, sem),
            pltpu.async_copy(fx_hbm.at[si_v], c3_v, sem),
            pltpu.async_copy(fy_hbm.at[si_v], c4_v, sem),
            pltpu.async_copy(fz_hbm.at[si_v], c5_v, sem),
        ]
        for cp in cps:
            cp.wait()
        for comp, dst in enumerate((ftx_o, fty_o, ftz_o, fsx_o, fsy_o, fsz_o)):
            pltpu.sync_copy(outs[comp], dst.at[pl.ds(base, CB)])
        return 0

    lax.fori_loop(0, NCHUNK, chunk, 0)


def _sc_gather(fx, fy, fz, ti, si):
    e = jax.ShapeDtypeStruct((N_EDGES,), jnp.float32)
    mesh = plsc.VectorSubcoreMesh(core_axis_name="c", subcore_axis_name="s")
    return pl.kernel(
        _gather_body,
        out_type=(e,) * 6,
        mesh=mesh,
        scratch_types=[
            pltpu.VMEM((CB,), jnp.int32),
            pltpu.VMEM((CB,), jnp.int32),
        ] + [pltpu.VMEM((CB,), jnp.float32)] * 6 + [pltpu.SemaphoreType.DMA],
    )(fx, fy, fz, ti, si)


def _mlp_body(rx, ry, rz, gtx, gty, gtz, gsx, gsy, gsz,
              w1t, b1c, w2t, b2c, prm, ox, oy, oz):
    def row(r):
        return r[...].reshape(1, BE)

    x, y, z = row(rx), row(ry), row(rz)
    d = jnp.sqrt(x * x + y * y + z * z)
    d = jnp.maximum(d, 1e-8)
    m = prm[0:1, 0:1]
    cd = prm[0:1, 1:2]
    rs = d - m
    rsq = rs * rs
    rq = rsq * rsq
    mind = d - cd
    feats = jnp.concatenate(
        [x, y, z, d, rsq, rq, mind,
         row(gtx), row(gty), row(gtz), row(gsx), row(gsy), row(gsz),
         jnp.zeros((3, BE), jnp.float32)], axis=0)
    h = jnp.tanh(
        jnp.dot(w1t[...], feats, preferred_element_type=jnp.float32) + b1c[...])
    v = jnp.dot(w2t[...], h, preferred_element_type=jnp.float32) + b2c[...]
    ox[...] = v[0:1, :].reshape(BE)
    oy[...] = v[1:2, :].reshape(BE)
    oz[...] = v[2:3, :].reshape(BE)


def _tc_mlp(comps, w1t, b1c, w2t, b2c, prm):
    grid = (N_EDGES // BE,)
    espec = pl.BlockSpec((BE,), lambda i: (i,))
    wspec = lambda shape: pl.BlockSpec(shape, lambda i: tuple(0 for _ in shape))
    e = jax.ShapeDtypeStruct((N_EDGES,), jnp.float32)
    return pl.pallas_call(
        _mlp_body,
        grid=grid,
        in_specs=[espec] * 9 + [wspec((32, 16)), wspec((32, 1)),
                                wspec((3, 32)), wspec((3, 1)), wspec((1, 2))],
        out_specs=[espec] * 3,
        out_shape=(e, e, e),
    )(*comps, w1t, b1c, w2t, b2c, prm)


def _scatter_body(ti_hbm, vx_hbm, vy_hbm, vz_hbm, out_hbm,
                  ti_v, vx_v, vy_v, vz_v, zb_v, shx, shy, shz):
    cid = lax.axis_index("c")
    sid = lax.axis_index("s")
    wid = sid * NC + cid

    def zb(i, _):
        zb_v[pl.ds(i * LANES, LANES)] = jnp.zeros((LANES,), jnp.float32)
        return 0

    lax.fori_loop(0, NPT // LANES, zb, 0)
    for sh in (shx, shy, shz):
        pltpu.sync_copy(zb_v, sh.at[pl.ds(sid * NPT, NPT)])
    plsc.subcore_barrier()

    def chunk(c, _):
        base = wid * PW + c * CB
        pltpu.sync_copy(ti_hbm.at[pl.ds(base, CB)], ti_v)
        pltpu.sync_copy(vx_hbm.at[pl.ds(base, CB)], vx_v)
        pltpu.sync_copy(vy_hbm.at[pl.ds(base, CB)], vy_v)
        pltpu.sync_copy(vz_hbm.at[pl.ds(base, CB)], vz_v)
        pltpu.sync_copy(vx_v, shx.at[ti_v], add=True)
        pltpu.sync_copy(vy_v, shy.at[ti_v], add=True)
        pltpu.sync_copy(vz_v, shz.at[ti_v], add=True)
        return 0

    lax.fori_loop(0, NCHUNK, chunk, 0)
    plsc.subcore_barrier()
    for comp, sh in enumerate((shx, shy, shz)):
        pltpu.sync_copy(sh.at[pl.ds(sid * NPT, NPT)], zb_v)
        pltpu.sync_copy(
            zb_v, out_hbm.at[pl.ds(cid * 3 * NP + comp * NP + sid * NPT, NPT)])


def _sc_scatter(ti, vx, vy, vz):
    mesh = plsc.VectorSubcoreMesh(core_axis_name="c", subcore_axis_name="s")
    return pl.kernel(
        _scatter_body,
        out_type=jax.ShapeDtypeStruct((NC * 3 * NP,), jnp.float32),
        mesh=mesh,
        scratch_types=[
            pltpu.VMEM((CB,), jnp.int32),
            pltpu.VMEM((CB,), jnp.float32),
            pltpu.VMEM((CB,), jnp.float32),
            pltpu.VMEM((CB,), jnp.float32),
            pltpu.VMEM((NPT,), jnp.float32),
            pltpu.VMEM_SHARED((NP,), jnp.float32),
            pltpu.VMEM_SHARED((NP,), jnp.float32),
            pltpu.VMEM_SHARED((NP,), jnp.float32),
        ],
    )(ti, vx, vy, vz)


def kernel(rel_selected, target_indices, source_indices, force, viscosity,
           W1, b1, W2, b2, median, contact_distance):
    ti = target_indices.astype(jnp.int32)
    si = source_indices.astype(jnp.int32)
    n = force.shape[0]
    fx = force[:, 0]
    fy = force[:, 1]
    fz = force[:, 2]
    relx = rel_selected[:, 0]
    rely = rel_selected[:, 1]
    relz = rel_selected[:, 2]
    w1t = jnp.concatenate([W1[:13], jnp.zeros((3, W1.shape[1]), jnp.float32)],
                          axis=0).T
    b1c = (b1 + viscosity * W1[13])[:, None]
    w2t = W2.T
    b2c = b2[:, None]
    prm = jnp.stack([median, contact_distance]).reshape(1, 2)

    gt = _sc_gather(fx, fy, fz, ti, si)
    vx, vy, vz = _tc_mlp((relx, rely, relz) + tuple(gt), w1t, b1c, w2t, b2c, prm)
    parts = _sc_scatter(ti, vx, vy, vz).reshape(NC, 3, NP)
    return (parts[0] + parts[1])[:, :n].T


# speedup vs baseline: 37.1152x; 1.2238x over previous
import functools

import jax
import jax.numpy as jnp
from jax import lax
from jax.experimental import pallas as pl
from jax.experimental.pallas import tpu as pltpu
from jax.experimental.pallas import tpu_sc as plsc

N_NODES = 100000
N_EDGES = 3200000
NC = 2
NS = 16
NW = NC * NS
LANES = 16

PW = N_EDGES // NW
CB = 2000
NCHUNK = PW // CB
NP = 100096
NPT = NP // NS

BE = 25600
DPAD = 8


def _gather_body(fp_hbm, ti_hbm, si_hbm,
                 ftx_o, fty_o, ftz_o, fsx_o, fsy_o, fsz_o,
                 ti_v, si_v, trows_v, srows_v, c0_v, c1_v, c2_v, c3_v, c4_v, c5_v,
                 sem):
    cid = lax.axis_index("c")
    sid = lax.axis_index("s")
    wid = sid * NC + cid
    outs = (c0_v, c1_v, c2_v, c3_v, c4_v, c5_v)

    def chunk(c, _):
        base = wid * PW + c * CB
        pltpu.sync_copy(ti_hbm.at[pl.ds(base, CB)], ti_v)
        pltpu.sync_copy(si_hbm.at[pl.ds(base, CB)], si_v)
        cps = [
            pltpu.async_copy(fp_hbm.at[ti_v], trows_v, sem),
            pltpu.async_copy(fp_hbm.at[si_v], srows_v, sem),
        ]
        for cp in cps:
            cp.wait()

        def deint(i, _):
            rows = jnp.arange(LANES, dtype=jnp.int32) + i * LANES
            for comp in range(3):
                col = jnp.full((LANES,), comp, dtype=jnp.int32)
                outs[comp][pl.ds(i * LANES, LANES)] = plsc.load_gather(
                    trows_v, [rows, col])
                outs[3 + comp][pl.ds(i * LANES, LANES)] = plsc.load_gather(
                    srows_v, [rows, col])
            return 0

        lax.fori_loop(0, CB // LANES, deint, 0)
        for comp, dst in enumerate((ftx_o, fty_o, ftz_o, fsx_o, fsy_o, fsz_o)):
            pltpu.sync_copy(outs[comp], dst.at[pl.ds(base, CB)])
        return 0

    lax.fori_loop(0, NCHUNK, chunk, 0)


def _sc_gather(fp, ti, si):
    e = jax.ShapeDtypeStruct((N_EDGES,), jnp.float32)
    mesh = plsc.VectorSubcoreMesh(core_axis_name="c", subcore_axis_name="s")
    return pl.kernel(
        _gather_body,
        out_type=(e,) * 6,
        mesh=mesh,
        scratch_types=[
            pltpu.VMEM((CB,), jnp.int32),
            pltpu.VMEM((CB,), jnp.int32),
            pltpu.VMEM((CB, DPAD), jnp.float32),
            pltpu.VMEM((CB, DPAD), jnp.float32),
        ] + [pltpu.VMEM((CB,), jnp.float32)] * 6 + [pltpu.SemaphoreType.DMA],
        compiler_params=pltpu.CompilerParams(use_tc_tiling_on_sc=False,
                                             needs_layout_passes=False),
    )(fp, ti, si)


def _mlp_body(rx, ry, rz, gtx, gty, gtz, gsx, gsy, gsz,
              w1t, b1c, w2t, b2c, prm, ox, oy, oz):
    def row(r):
        return r[...].reshape(1, BE)

    x, y, z = row(rx), row(ry), row(rz)
    d = jnp.sqrt(x * x + y * y + z * z)
    d = jnp.maximum(d, 1e-8)
    m = prm[0:1, 0:1]
    cd = prm[0:1, 1:2]
    rs = d - m
    rsq = rs * rs
    rq = rsq * rsq
    mind = d - cd
    feats = jnp.concatenate(
        [x, y, z, d, rsq, rq, mind,
         row(gtx), row(gty), row(gtz), row(gsx), row(gsy), row(gsz),
         jnp.zeros((3, BE), jnp.float32)], axis=0)
    h = jnp.tanh(
        jnp.dot(w1t[...], feats, preferred_element_type=jnp.float32) + b1c[...])
    v = jnp.dot(w2t[...], h, preferred_element_type=jnp.float32) + b2c[...]
    ox[...] = v[0:1, :].reshape(BE)
    oy[...] = v[1:2, :].reshape(BE)
    oz[...] = v[2:3, :].reshape(BE)


def _tc_mlp(comps, w1t, b1c, w2t, b2c, prm):
    grid = (N_EDGES // BE,)
    espec = pl.BlockSpec((BE,), lambda i: (i,))
    wspec = lambda shape: pl.BlockSpec(shape, lambda i: tuple(0 for _ in shape))
    e = jax.ShapeDtypeStruct((N_EDGES,), jnp.float32)
    return pl.pallas_call(
        _mlp_body,
        grid=grid,
        in_specs=[espec] * 9 + [wspec((32, 16)), wspec((32, 1)),
                                wspec((3, 32)), wspec((3, 1)), wspec((1, 2))],
        out_specs=[espec] * 3,
        out_shape=(e, e, e),
    )(*comps, w1t, b1c, w2t, b2c, prm)


def _scatter_body(ti_hbm, vx_hbm, vy_hbm, vz_hbm, out_hbm,
                  ti_v, vx_v, vy_v, vz_v, zb_v, shx, shy, shz):
    cid = lax.axis_index("c")
    sid = lax.axis_index("s")
    wid = sid * NC + cid

    def zb(i, _):
        zb_v[pl.ds(i * LANES, LANES)] = jnp.zeros((LANES,), jnp.float32)
        return 0

    lax.fori_loop(0, NPT // LANES, zb, 0)
    for sh in (shx, shy, shz):
        pltpu.sync_copy(zb_v, sh.at[pl.ds(sid * NPT, NPT)])
    plsc.subcore_barrier()

    def chunk(c, _):
        base = wid * PW + c * CB
        pltpu.sync_copy(ti_hbm.at[pl.ds(base, CB)], ti_v)
        pltpu.sync_copy(vx_hbm.at[pl.ds(base, CB)], vx_v)
        pltpu.sync_copy(vy_hbm.at[pl.ds(base, CB)], vy_v)
        pltpu.sync_copy(vz_hbm.at[pl.ds(base, CB)], vz_v)
        pltpu.sync_copy(vx_v, shx.at[ti_v], add=True)
        pltpu.sync_copy(vy_v, shy.at[ti_v], add=True)
        pltpu.sync_copy(vz_v, shz.at[ti_v], add=True)
        return 0

    lax.fori_loop(0, NCHUNK, chunk, 0)
    plsc.subcore_barrier()
    for comp, sh in enumerate((shx, shy, shz)):
        pltpu.sync_copy(sh.at[pl.ds(sid * NPT, NPT)], zb_v)
        pltpu.sync_copy(
            zb_v, out_hbm.at[pl.ds(cid * 3 * NP + comp * NP + sid * NPT, NPT)])


def _sc_scatter(ti, vx, vy, vz):
    mesh = plsc.VectorSubcoreMesh(core_axis_name="c", subcore_axis_name="s")
    return pl.kernel(
        _scatter_body,
        out_type=jax.ShapeDtypeStruct((NC * 3 * NP,), jnp.float32),
        mesh=mesh,
        scratch_types=[
            pltpu.VMEM((CB,), jnp.int32),
            pltpu.VMEM((CB,), jnp.float32),
            pltpu.VMEM((CB,), jnp.float32),
            pltpu.VMEM((CB,), jnp.float32),
            pltpu.VMEM((NPT,), jnp.float32),
            pltpu.VMEM_SHARED((NP,), jnp.float32),
            pltpu.VMEM_SHARED((NP,), jnp.float32),
            pltpu.VMEM_SHARED((NP,), jnp.float32),
        ],
    )(ti, vx, vy, vz)


def kernel(rel_selected, target_indices, source_indices, force, viscosity,
           W1, b1, W2, b2, median, contact_distance):
    ti = target_indices.astype(jnp.int32)
    si = source_indices.astype(jnp.int32)
    n = force.shape[0]
    fp = jnp.concatenate([force, jnp.zeros((n, DPAD - 3), jnp.float32)], axis=1)
    relx = rel_selected[:, 0]
    rely = rel_selected[:, 1]
    relz = rel_selected[:, 2]
    w1t = jnp.concatenate([W1[:13], jnp.zeros((3, W1.shape[1]), jnp.float32)],
                          axis=0).T
    b1c = (b1 + viscosity * W1[13])[:, None]
    w2t = W2.T
    b2c = b2[:, None]
    prm = jnp.stack([median, contact_distance]).reshape(1, 2)

    gt = _sc_gather(fp, ti, si)
    vx, vy, vz = _tc_mlp((relx, rely, relz) + tuple(gt), w1t, b1c, w2t, b2c, prm)
    parts = _sc_scatter(ti, vx, vy, vz).reshape(NC, 3, NP)
    return (parts[0] + parts[1])[:, :n].T


# speedup vs baseline: 40.5957x; 1.0938x over previous
import functools

import jax
import jax.numpy as jnp
from jax import lax
from jax.experimental import pallas as pl
from jax.experimental.pallas import tpu as pltpu
from jax.experimental.pallas import tpu_sc as plsc

N_NODES = 100000
N_EDGES = 3200000
NC = 2
NS = 16
NW = NC * NS
LANES = 16

PW = N_EDGES // NW
CB = 2000
NCHUNK = PW // CB
NP = 100096
NPT = NP // NS

BE = 25600
DPAD = 8


def _gather_body(fp_hbm, ti_hbm, si_hbm,
                 ftx_o, fty_o, ftz_o, fsx_o, fsy_o, fsz_o,
                 ti_v, si_v, trows_v, srows_v, couts_v,
                 semi, semg, semo):
    cid = lax.axis_index("c")
    sid = lax.axis_index("s")
    wid = sid * NC + cid
    dsts = (ftx_o, fty_o, ftz_o, fsx_o, fsy_o, fsz_o)

    def cbase(c):
        return wid * PW + jnp.minimum(c, NCHUNK - 1) * CB

    def issue_idx(c, b):
        base = cbase(c)
        pltpu.async_copy(ti_hbm.at[pl.ds(base, CB)], ti_v.at[b], semi)
        pltpu.async_copy(si_hbm.at[pl.ds(base, CB)], si_v.at[b], semi)

    def drain_idx(b):
        pltpu.make_async_copy(ti_hbm.at[pl.ds(0, CB)], ti_v.at[b], semi).wait()
        pltpu.make_async_copy(si_hbm.at[pl.ds(0, CB)], si_v.at[b], semi).wait()

    def issue_gather(b):
        pltpu.async_copy(fp_hbm.at[ti_v.at[b]], trows_v.at[b], semg)
        pltpu.async_copy(fp_hbm.at[si_v.at[b]], srows_v.at[b], semg)

    def drain_gather(b):
        pltpu.make_async_copy(fp_hbm.at[ti_v.at[b]], trows_v.at[b], semg).wait()
        pltpu.make_async_copy(fp_hbm.at[si_v.at[b]], srows_v.at[b], semg).wait()

    def deint(b):
        def step(i, _):
            rows = jnp.arange(LANES, dtype=jnp.int32) + i * LANES
            for comp in range(3):
                col = jnp.full((LANES,), comp, dtype=jnp.int32)
                couts_v[b, comp, pl.ds(i * LANES, LANES)] = plsc.load_gather(
                    trows_v.at[b], [rows, col])
                couts_v[b, 3 + comp, pl.ds(i * LANES, LANES)] = plsc.load_gather(
                    srows_v.at[b], [rows, col])
            return 0

        lax.fori_loop(0, CB // LANES, step, 0)

    def issue_out(c, b):
        base = cbase(c)
        for comp in range(6):
            pltpu.async_copy(couts_v.at[b, comp],
                             dsts[comp].at[pl.ds(base, CB)], semo)

    def drain_out(b):
        for comp in range(6):
            pltpu.make_async_copy(couts_v.at[b, comp],
                                  dsts[comp].at[pl.ds(0, CB)], semo).wait()

    def run_chunk(c, b, first):
        drain_idx(b)
        issue_gather(b)
        issue_idx(c + 1, 1 - b)
        drain_gather(b)
        if not first:
            drain_out(b)
        deint(b)
        issue_out(c, b)

    issue_idx(0, 0)
    run_chunk(0, 0, True)
    run_chunk(1, 1, True)

    def pair(j, _):
        run_chunk(2 * j, 0, False)
        run_chunk(2 * j + 1, 1, False)
        return 0

    lax.fori_loop(1, NCHUNK // 2, pair, 0)
    drain_out(0)
    drain_out(1)
    drain_idx(0)


def _sc_gather(fp, ti, si):
    e = jax.ShapeDtypeStruct((N_EDGES,), jnp.float32)
    mesh = plsc.VectorSubcoreMesh(core_axis_name="c", subcore_axis_name="s")
    return pl.kernel(
        _gather_body,
        out_type=(e,) * 6,
        mesh=mesh,
        scratch_types=[
            pltpu.VMEM((2, CB), jnp.int32),
            pltpu.VMEM((2, CB), jnp.int32),
            pltpu.VMEM((2, CB, DPAD), jnp.float32),
            pltpu.VMEM((2, CB, DPAD), jnp.float32),
            pltpu.VMEM((2, 6, CB), jnp.float32),
            pltpu.SemaphoreType.DMA,
            pltpu.SemaphoreType.DMA,
            pltpu.SemaphoreType.DMA,
        ],
        compiler_params=pltpu.CompilerParams(use_tc_tiling_on_sc=False,
                                             needs_layout_passes=False),
    )(fp, ti, si)


def _mlp_body(rx, ry, rz, gtx, gty, gtz, gsx, gsy, gsz,
              w1t, b1c, w2t, b2c, prm, ox, oy, oz):
    def row(r):
        return r[...].reshape(1, BE)

    x, y, z = row(rx), row(ry), row(rz)
    d = jnp.sqrt(x * x + y * y + z * z)
    d = jnp.maximum(d, 1e-8)
    m = prm[0:1, 0:1]
    cd = prm[0:1, 1:2]
    rs = d - m
    rsq = rs * rs
    rq = rsq * rsq
    mind = d - cd
    feats = jnp.concatenate(
        [x, y, z, d, rsq, rq, mind,
         row(gtx), row(gty), row(gtz), row(gsx), row(gsy), row(gsz),
         jnp.zeros((3, BE), jnp.float32)], axis=0)
    h = jnp.tanh(
        jnp.dot(w1t[...], feats, preferred_element_type=jnp.float32) + b1c[...])
    v = jnp.dot(w2t[...], h, preferred_element_type=jnp.float32) + b2c[...]
    ox[...] = v[0:1, :].reshape(BE)
    oy[...] = v[1:2, :].reshape(BE)
    oz[...] = v[2:3, :].reshape(BE)


def _tc_mlp(comps, w1t, b1c, w2t, b2c, prm):
    grid = (N_EDGES // BE,)
    espec = pl.BlockSpec((BE,), lambda i: (i,))
    wspec = lambda shape: pl.BlockSpec(shape, lambda i: tuple(0 for _ in shape))
    e = jax.ShapeDtypeStruct((N_EDGES,), jnp.float32)
    return pl.pallas_call(
        _mlp_body,
        grid=grid,
        in_specs=[espec] * 9 + [wspec((32, 16)), wspec((32, 1)),
                                wspec((3, 32)), wspec((3, 1)), wspec((1, 2))],
        out_specs=[espec] * 3,
        out_shape=(e, e, e),
    )(*comps, w1t, b1c, w2t, b2c, prm)


def _scatter_body(ti_hbm, vx_hbm, vy_hbm, vz_hbm, out_hbm,
                  ti_v, vx_v, vy_v, vz_v, zb_v, shx, shy, shz):
    cid = lax.axis_index("c")
    sid = lax.axis_index("s")
    wid = sid * NC + cid

    def zb(i, _):
        zb_v[pl.ds(i * LANES, LANES)] = jnp.zeros((LANES,), jnp.float32)
        return 0

    lax.fori_loop(0, NPT // LANES, zb, 0)
    for sh in (shx, shy, shz):
        pltpu.sync_copy(zb_v, sh.at[pl.ds(sid * NPT, NPT)])
    plsc.subcore_barrier()

    def chunk(c, _):
        base = wid * PW + c * CB
        pltpu.sync_copy(ti_hbm.at[pl.ds(base, CB)], ti_v)
        pltpu.sync_copy(vx_hbm.at[pl.ds(base, CB)], vx_v)
        pltpu.sync_copy(vy_hbm.at[pl.ds(base, CB)], vy_v)
        pltpu.sync_copy(vz_hbm.at[pl.ds(base, CB)], vz_v)
        pltpu.sync_copy(vx_v, shx.at[ti_v], add=True)
        pltpu.sync_copy(vy_v, shy.at[ti_v], add=True)
        pltpu.sync_copy(vz_v, shz.at[ti_v], add=True)
        return 0

    lax.fori_loop(0, NCHUNK, chunk, 0)
    plsc.subcore_barrier()
    for comp, sh in enumerate((shx, shy, shz)):
        pltpu.sync_copy(sh.at[pl.ds(sid * NPT, NPT)], zb_v)
        pltpu.sync_copy(
            zb_v, out_hbm.at[pl.ds(cid * 3 * NP + comp * NP + sid * NPT, NPT)])


def _sc_scatter(ti, vx, vy, vz):
    mesh = plsc.VectorSubcoreMesh(core_axis_name="c", subcore_axis_name="s")
    return pl.kernel(
        _scatter_body,
        out_type=jax.ShapeDtypeStruct((NC * 3 * NP,), jnp.float32),
        mesh=mesh,
        scratch_types=[
            pltpu.VMEM((CB,), jnp.int32),
            pltpu.VMEM((CB,), jnp.float32),
            pltpu.VMEM((CB,), jnp.float32),
            pltpu.VMEM((CB,), jnp.float32),
            pltpu.VMEM((NPT,), jnp.float32),
            pltpu.VMEM_SHARED((NP,), jnp.float32),
            pltpu.VMEM_SHARED((NP,), jnp.float32),
            pltpu.VMEM_SHARED((NP,), jnp.float32),
        ],
    )(ti, vx, vy, vz)


def kernel(rel_selected, target_indices, source_indices, force, viscosity,
           W1, b1, W2, b2, median, contact_distance):
    ti = target_indices.astype(jnp.int32)
    si = source_indices.astype(jnp.int32)
    n = force.shape[0]
    fp = jnp.concatenate([force, jnp.zeros((n, DPAD - 3), jnp.float32)], axis=1)
    relx = rel_selected[:, 0]
    rely = rel_selected[:, 1]
    relz = rel_selected[:, 2]
    w1t = jnp.concatenate([W1[:13], jnp.zeros((3, W1.shape[1]), jnp.float32)],
                          axis=0).T
    b1c = (b1 + viscosity * W1[13])[:, None]
    w2t = W2.T
    b2c = b2[:, None]
    prm = jnp.stack([median, contact_distance]).reshape(1, 2)

    gt = _sc_gather(fp, ti, si)
    vx, vy, vz = _tc_mlp((relx, rely, relz) + tuple(gt), w1t, b1c, w2t, b2c, prm)
    parts = _sc_scatter(ti, vx, vy, vz).reshape(NC, 3, NP)
    return (parts[0] + parts[1])[:, :n].T


# speedup vs baseline: 54.9461x; 1.3535x over previous
import functools

import jax
import jax.numpy as jnp
from jax import lax
from jax.experimental import pallas as pl
from jax.experimental.pallas import tpu as pltpu
from jax.experimental.pallas import tpu_sc as plsc

N_NODES = 100000
N_EDGES = 3200000
NC = 2
NS = 16
NW = NC * NS
LANES = 16

PW = N_EDGES // NW
CB = 2000
NCHUNK = PW // CB
NP = 100096
NPT = NP // NS

BE = 25600
DPAD = 8


def _gather_body(fp_hbm, ti_hbm, si_hbm,
                 ftx_o, fty_o, ftz_o, fsx_o, fsy_o, fsz_o,
                 ti_v, si_v, trows_v, srows_v, couts_v,
                 semi, semg, semo):
    cid = lax.axis_index("c")
    sid = lax.axis_index("s")
    wid = sid * NC + cid
    dsts = (ftx_o, fty_o, ftz_o, fsx_o, fsy_o, fsz_o)

    def cbase(c):
        return wid * PW + jnp.minimum(c, NCHUNK - 1) * CB

    def issue_idx(c, b):
        base = cbase(c)
        pltpu.async_copy(ti_hbm.at[pl.ds(base, CB)], ti_v.at[b], semi)
        pltpu.async_copy(si_hbm.at[pl.ds(base, CB)], si_v.at[b], semi)

    def drain_idx(b):
        pltpu.make_async_copy(ti_hbm.at[pl.ds(0, CB)], ti_v.at[b], semi).wait()
        pltpu.make_async_copy(si_hbm.at[pl.ds(0, CB)], si_v.at[b], semi).wait()

    def issue_gather(b):
        pltpu.async_copy(fp_hbm.at[ti_v.at[b]], trows_v.at[b], semg)
        pltpu.async_copy(fp_hbm.at[si_v.at[b]], srows_v.at[b], semg)

    def drain_gather(b):
        pltpu.make_async_copy(fp_hbm.at[ti_v.at[b]], trows_v.at[b], semg).wait()
        pltpu.make_async_copy(fp_hbm.at[si_v.at[b]], srows_v.at[b], semg).wait()

    def deint(b):
        @plsc.parallel_loop(0, CB // LANES, step=1, unroll=5)
        def step(i):
            rows = jnp.arange(LANES, dtype=jnp.int32) + i * LANES
            for comp in range(3):
                col = jnp.full((LANES,), comp, dtype=jnp.int32)
                couts_v[b, comp, pl.ds(i * LANES, LANES)] = plsc.load_gather(
                    trows_v.at[b], [rows, col])
                couts_v[b, 3 + comp, pl.ds(i * LANES, LANES)] = plsc.load_gather(
                    srows_v.at[b], [rows, col])

    def issue_out(c, b):
        base = cbase(c)
        for comp in range(6):
            pltpu.async_copy(couts_v.at[b, comp],
                             dsts[comp].at[pl.ds(base, CB)], semo)

    def drain_out(b):
        for comp in range(6):
            pltpu.make_async_copy(couts_v.at[b, comp],
                                  dsts[comp].at[pl.ds(0, CB)], semo).wait()

    def run_chunk(c, b, first):
        drain_idx(b)
        issue_gather(b)
        issue_idx(c + 1, 1 - b)
        drain_gather(b)
        if not first:
            drain_out(b)
        deint(b)
        issue_out(c, b)

    issue_idx(0, 0)
    run_chunk(0, 0, True)
    run_chunk(1, 1, True)

    def pair(j, _):
        run_chunk(2 * j, 0, False)
        run_chunk(2 * j + 1, 1, False)
        return 0

    lax.fori_loop(1, NCHUNK // 2, pair, 0)
    drain_out(0)
    drain_out(1)
    drain_idx(0)


def _sc_gather(fp, ti, si):
    e = jax.ShapeDtypeStruct((N_EDGES,), jnp.float32)
    mesh = plsc.VectorSubcoreMesh(core_axis_name="c", subcore_axis_name="s")
    return pl.kernel(
        _gather_body,
        out_type=(e,) * 6,
        mesh=mesh,
        scratch_types=[
            pltpu.VMEM((2, CB), jnp.int32),
            pltpu.VMEM((2, CB), jnp.int32),
            pltpu.VMEM((2, CB, DPAD), jnp.float32),
            pltpu.VMEM((2, CB, DPAD), jnp.float32),
            pltpu.VMEM((2, 6, CB), jnp.float32),
            pltpu.SemaphoreType.DMA,
            pltpu.SemaphoreType.DMA,
            pltpu.SemaphoreType.DMA,
        ],
        compiler_params=pltpu.CompilerParams(use_tc_tiling_on_sc=False,
                                             needs_layout_passes=False),
    )(fp, ti, si)


def _mlp_body(rx, ry, rz, gtx, gty, gtz, gsx, gsy, gsz,
              w1t, b1c, w2t, b2c, prm, ox, oy, oz):
    def row(r):
        return r[...].reshape(1, BE)

    x, y, z = row(rx), row(ry), row(rz)
    d = jnp.sqrt(x * x + y * y + z * z)
    d = jnp.maximum(d, 1e-8)
    m = prm[0:1, 0:1]
    cd = prm[0:1, 1:2]
    rs = d - m
    rsq = rs * rs
    rq = rsq * rsq
    mind = d - cd
    feats = jnp.concatenate(
        [x, y, z, d, rsq, rq, mind,
         row(gtx), row(gty), row(gtz), row(gsx), row(gsy), row(gsz),
         jnp.zeros((3, BE), jnp.float32)], axis=0)
    h = jnp.tanh(
        jnp.dot(w1t[...], feats, preferred_element_type=jnp.float32) + b1c[...])
    v = jnp.dot(w2t[...], h, preferred_element_type=jnp.float32) + b2c[...]
    ox[...] = v[0:1, :].reshape(BE)
    oy[...] = v[1:2, :].reshape(BE)
    oz[...] = v[2:3, :].reshape(BE)


def _tc_mlp(comps, w1t, b1c, w2t, b2c, prm):
    grid = (N_EDGES // BE,)
    espec = pl.BlockSpec((BE,), lambda i: (i,))
    wspec = lambda shape: pl.BlockSpec(shape, lambda i: tuple(0 for _ in shape))
    e = jax.ShapeDtypeStruct((N_EDGES,), jnp.float32)
    return pl.pallas_call(
        _mlp_body,
        grid=grid,
        in_specs=[espec] * 9 + [wspec((32, 16)), wspec((32, 1)),
                                wspec((3, 32)), wspec((3, 1)), wspec((1, 2))],
        out_specs=[espec] * 3,
        out_shape=(e, e, e),
    )(*comps, w1t, b1c, w2t, b2c, prm)


def _scatter_body(ti_hbm, vx_hbm, vy_hbm, vz_hbm, out_hbm,
                  ti_v, vv_v, zb_v, shx, shy, shz):
    cid = lax.axis_index("c")
    sid = lax.axis_index("s")
    wid = sid * NC + cid
    shs = (shx, shy, shz)
    vins = (vx_hbm, vy_hbm, vz_hbm)

    def zb(i, _):
        zb_v[pl.ds(i * LANES, LANES)] = jnp.zeros((LANES,), jnp.float32)
        return 0

    lax.fori_loop(0, NPT // LANES, zb, 0)
    for sh in shs:
        pltpu.sync_copy(zb_v, sh.at[pl.ds(sid * NPT, NPT)])
    plsc.subcore_barrier()

    def cbase(c):
        return wid * PW + jnp.minimum(c, NCHUNK - 1) * CB

    def issue_in(c, b, semi):
        base = cbase(c)
        pltpu.async_copy(ti_hbm.at[pl.ds(base, CB)], ti_v.at[b], semi)
        for comp in range(3):
            pltpu.async_copy(vins[comp].at[pl.ds(base, CB)],
                             vv_v.at[b, comp], semi)

    def drain_in(b, semi):
        pltpu.make_async_copy(ti_hbm.at[pl.ds(0, CB)], ti_v.at[b], semi).wait()
        for comp in range(3):
            pltpu.make_async_copy(vins[comp].at[pl.ds(0, CB)],
                                  vv_v.at[b, comp], semi).wait()

    def issue_add(b, sema):
        for comp in range(3):
            pltpu.async_copy(vv_v.at[b, comp], shs[comp].at[ti_v.at[b]],
                             sema, add=True)

    def drain_add(b, sema):
        for comp in range(3):
            pltpu.make_async_copy(vv_v.at[b, comp],
                                  shs[comp].at[ti_v.at[b]], sema).wait()

    def run_chunk(c, b, first, semi, sema):
        drain_in(b, semi)
        if not first:
            drain_add(1 - b, sema)
        issue_in(c + 1, 1 - b, semi)
        issue_add(b, sema)

    def scatter_phase(semi, sema):
        issue_in(0, 0, semi)
        run_chunk(0, 0, True, semi, sema)
        run_chunk(1, 1, False, semi, sema)

        def pair(j, _):
            run_chunk(2 * j, 0, False, semi, sema)
            run_chunk(2 * j + 1, 1, False, semi, sema)
            return 0

        lax.fori_loop(1, NCHUNK // 2, pair, 0)
        drain_add(1, sema)
        drain_in(0, semi)

    pl.run_scoped(scatter_phase,
                  pltpu.SemaphoreType.DMA, pltpu.SemaphoreType.DMA)
    plsc.subcore_barrier()
    for comp, sh in enumerate(shs):
        pltpu.sync_copy(sh.at[pl.ds(sid * NPT, NPT)], zb_v)
        pltpu.sync_copy(
            zb_v, out_hbm.at[pl.ds(cid * 3 * NP + comp * NP + sid * NPT, NPT)])


def _sc_scatter(ti, vx, vy, vz):
    mesh = plsc.VectorSubcoreMesh(core_axis_name="c", subcore_axis_name="s")
    return pl.kernel(
        _scatter_body,
        out_type=jax.ShapeDtypeStruct((NC * 3 * NP,), jnp.float32),
        mesh=mesh,
        scratch_types=[
            pltpu.VMEM((2, CB), jnp.int32),
            pltpu.VMEM((2, 3, CB), jnp.float32),
            pltpu.VMEM((NPT,), jnp.float32),
            pltpu.VMEM_SHARED((NP,), jnp.float32),
            pltpu.VMEM_SHARED((NP,), jnp.float32),
            pltpu.VMEM_SHARED((NP,), jnp.float32),
        ],
        compiler_params=pltpu.CompilerParams(use_tc_tiling_on_sc=False,
                                             needs_layout_passes=False),
    )(ti, vx, vy, vz)


def kernel(rel_selected, target_indices, source_indices, force, viscosity,
           W1, b1, W2, b2, median, contact_distance):
    ti = target_indices.astype(jnp.int32)
    si = source_indices.astype(jnp.int32)
    n = force.shape[0]
    fp = jnp.concatenate([force, jnp.zeros((n, DPAD - 3), jnp.float32)], axis=1)
    relx = rel_selected[:, 0]
    rely = rel_selected[:, 1]
    relz = rel_selected[:, 2]
    w1t = jnp.concatenate([W1[:13], jnp.zeros((3, W1.shape[1]), jnp.float32)],
                          axis=0).T
    b1c = (b1 + viscosity * W1[13])[:, None]
    w2t = W2.T
    b2c = b2[:, None]
    prm = jnp.stack([median, contact_distance]).reshape(1, 2)

    gt = _sc_gather(fp, ti, si)
    vx, vy, vz = _tc_mlp((relx, rely, relz) + tuple(gt), w1t, b1c, w2t, b2c, prm)
    parts = _sc_scatter(ti, vx, vy, vz).reshape(NC, 3, NP)
    return (parts[0] + parts[1])[:, :n].T


# speedup vs baseline: 59.2654x; 1.0786x over previous
import functools

import jax
import jax.numpy as jnp
from jax import lax
from jax.experimental import pallas as pl
from jax.experimental.pallas import tpu as pltpu
from jax.experimental.pallas import tpu_sc as plsc

N_NODES = 100000
N_EDGES = 3200000
NC = 2
NS = 16
NW = NC * NS
LANES = 16

PW = N_EDGES // NW
CB = 2000
NCHUNK = PW // CB
GCB = 800
GNCH = PW // GCB
NB = 5
NP = 100096
NPT = NP // NS

BE = 25600
DPAD = 8


def _gather_body(fp_hbm, ti_hbm, si_hbm,
                 ftx_o, fty_o, ftz_o, fsx_o, fsy_o, fsz_o,
                 ti_v, si_v, trows_v, srows_v, couts_v,
                 semi, semg, semo):
    cid = lax.axis_index("c")
    sid = lax.axis_index("s")
    wid = sid * NC + cid
    dsts = (ftx_o, fty_o, ftz_o, fsx_o, fsy_o, fsz_o)

    def cbase(c):
        return wid * PW + jnp.minimum(c, GNCH - 1) * GCB

    def issue_idx(c, b):
        base = cbase(c)
        pltpu.async_copy(ti_hbm.at[pl.ds(base, GCB)], ti_v.at[b], semi.at[b])
        pltpu.async_copy(si_hbm.at[pl.ds(base, GCB)], si_v.at[b], semi.at[b])

    def drain_idx(b):
        pltpu.make_async_copy(
            ti_hbm.at[pl.ds(0, GCB)], ti_v.at[b], semi.at[b]).wait()
        pltpu.make_async_copy(
            si_hbm.at[pl.ds(0, GCB)], si_v.at[b], semi.at[b]).wait()

    def issue_gather(b):
        pltpu.async_copy(fp_hbm.at[ti_v.at[b]], trows_v.at[b], semg.at[b])
        pltpu.async_copy(fp_hbm.at[si_v.at[b]], srows_v.at[b], semg.at[b])

    def drain_gather(b):
        pltpu.make_async_copy(
            fp_hbm.at[ti_v.at[b]], trows_v.at[b], semg.at[b]).wait()
        pltpu.make_async_copy(
            fp_hbm.at[si_v.at[b]], srows_v.at[b], semg.at[b]).wait()

    def deint(b):
        @plsc.parallel_loop(0, GCB // LANES, step=1, unroll=5)
        def step(i):
            rows = jnp.arange(LANES, dtype=jnp.int32) + i * LANES
            for comp in range(3):
                col = jnp.full((LANES,), comp, dtype=jnp.int32)
                couts_v[b, comp, pl.ds(i * LANES, LANES)] = plsc.load_gather(
                    trows_v.at[b], [rows, col])
                couts_v[b, 3 + comp, pl.ds(i * LANES, LANES)] = plsc.load_gather(
                    srows_v.at[b], [rows, col])

    def issue_out(c, b):
        base = cbase(c)
        for comp in range(6):
            pltpu.async_copy(couts_v.at[b, comp],
                             dsts[comp].at[pl.ds(base, GCB)], semo.at[b])

    def drain_out(b):
        for comp in range(6):
            pltpu.make_async_copy(couts_v.at[b, comp],
                                  dsts[comp].at[pl.ds(0, GCB)],
                                  semo.at[b]).wait()

    def body(c, b, first):
        drain_gather(b)
        if not first:
            drain_out(b)
        deint(b)
        issue_out(c, b)
        b2 = (b + 2) % NB
        drain_idx(b2)
        issue_gather(b2)
        issue_idx(c + 4, (b + 4) % NB)

    for c0 in range(4):
        issue_idx(c0, c0)
    drain_idx(0)
    issue_gather(0)
    drain_idx(1)
    issue_gather(1)
    for c0 in range(NB):
        body(c0, c0, True)

    def group(j, _):
        for b0 in range(NB):
            body(NB * j + b0, b0, False)
        return 0

    lax.fori_loop(1, GNCH // NB, group, 0)
    drain_gather(0)
    drain_gather(1)
    drain_idx(2)
    drain_idx(3)
    for b0 in range(NB):
        drain_out(b0)


def _sc_gather(fp, ti, si):
    e = jax.ShapeDtypeStruct((N_EDGES,), jnp.float32)
    mesh = plsc.VectorSubcoreMesh(core_axis_name="c", subcore_axis_name="s")
    return pl.kernel(
        _gather_body,
        out_type=(e,) * 6,
        mesh=mesh,
        scratch_types=[
            pltpu.VMEM((NB, GCB), jnp.int32),
            pltpu.VMEM((NB, GCB), jnp.int32),
            pltpu.VMEM((NB, GCB, DPAD), jnp.float32),
            pltpu.VMEM((NB, GCB, DPAD), jnp.float32),
            pltpu.VMEM((NB, 6, GCB), jnp.float32),
            pltpu.SemaphoreType.DMA((NB,)),
            pltpu.SemaphoreType.DMA((NB,)),
            pltpu.SemaphoreType.DMA((NB,)),
        ],
        compiler_params=pltpu.CompilerParams(use_tc_tiling_on_sc=False,
                                             needs_layout_passes=False),
    )(fp, ti, si)


def _mlp_body(rx, ry, rz, gtx, gty, gtz, gsx, gsy, gsz,
              w1t, b1c, w2t, b2c, prm, ox, oy, oz):
    def row(r):
        return r[...].reshape(1, BE)

    x, y, z = row(rx), row(ry), row(rz)
    d = jnp.sqrt(x * x + y * y + z * z)
    d = jnp.maximum(d, 1e-8)
    m = prm[0:1, 0:1]
    cd = prm[0:1, 1:2]
    rs = d - m
    rsq = rs * rs
    rq = rsq * rsq
    mind = d - cd
    feats = jnp.concatenate(
        [x, y, z, d, rsq, rq, mind,
         row(gtx), row(gty), row(gtz), row(gsx), row(gsy), row(gsz),
         jnp.zeros((3, BE), jnp.float32)], axis=0)
    h = jnp.tanh(
        jnp.dot(w1t[...], feats, preferred_element_type=jnp.float32) + b1c[...])
    v = jnp.dot(w2t[...], h, preferred_element_type=jnp.float32) + b2c[...]
    ox[...] = v[0:1, :].reshape(BE)
    oy[...] = v[1:2, :].reshape(BE)
    oz[...] = v[2:3, :].reshape(BE)


def _tc_mlp(comps, w1t, b1c, w2t, b2c, prm):
    grid = (N_EDGES // BE,)
    espec = pl.BlockSpec((BE,), lambda i: (i,))
    wspec = lambda shape: pl.BlockSpec(shape, lambda i: tuple(0 for _ in shape))
    e = jax.ShapeDtypeStruct((N_EDGES,), jnp.float32)
    return pl.pallas_call(
        _mlp_body,
        grid=grid,
        in_specs=[espec] * 9 + [wspec((32, 16)), wspec((32, 1)),
                                wspec((3, 32)), wspec((3, 1)), wspec((1, 2))],
        out_specs=[espec] * 3,
        out_shape=(e, e, e),
    )(*comps, w1t, b1c, w2t, b2c, prm)


def _scatter_body(ti_hbm, vx_hbm, vy_hbm, vz_hbm, out_hbm,
                  ti_v, vv_v, zb_v, shx, shy, shz):
    cid = lax.axis_index("c")
    sid = lax.axis_index("s")
    wid = sid * NC + cid
    shs = (shx, shy, shz)
    vins = (vx_hbm, vy_hbm, vz_hbm)

    def zb(i, _):
        zb_v[pl.ds(i * LANES, LANES)] = jnp.zeros((LANES,), jnp.float32)
        return 0

    lax.fori_loop(0, NPT // LANES, zb, 0)
    for sh in shs:
        pltpu.sync_copy(zb_v, sh.at[pl.ds(sid * NPT, NPT)])
    plsc.subcore_barrier()

    def cbase(c):
        return wid * PW + jnp.minimum(c, NCHUNK - 1) * CB

    def issue_in(c, b, semi):
        base = cbase(c)
        pltpu.async_copy(ti_hbm.at[pl.ds(base, CB)], ti_v.at[b], semi)
        for comp in range(3):
            pltpu.async_copy(vins[comp].at[pl.ds(base, CB)],
                             vv_v.at[b, comp], semi)

    def drain_in(b, semi):
        pltpu.make_async_copy(ti_hbm.at[pl.ds(0, CB)], ti_v.at[b], semi).wait()
        for comp in range(3):
            pltpu.make_async_copy(vins[comp].at[pl.ds(0, CB)],
                                  vv_v.at[b, comp], semi).wait()

    def issue_add(b, sema):
        for comp in range(3):
            pltpu.async_copy(vv_v.at[b, comp], shs[comp].at[ti_v.at[b]],
                             sema, add=True)

    def drain_add(b, sema):
        for comp in range(3):
            pltpu.make_async_copy(vv_v.at[b, comp],
                                  shs[comp].at[ti_v.at[b]], sema).wait()

    def run_chunk(c, b, first, semi, sema):
        drain_in(b, semi)
        if not first:
            drain_add(1 - b, sema)
        issue_in(c + 1, 1 - b, semi)
        issue_add(b, sema)

    def scatter_phase(semi, sema):
        issue_in(0, 0, semi)
        run_chunk(0, 0, True, semi, sema)
        run_chunk(1, 1, False, semi, sema)

        def pair(j, _):
            run_chunk(2 * j, 0, False, semi, sema)
            run_chunk(2 * j + 1, 1, False, semi, sema)
            return 0

        lax.fori_loop(1, NCHUNK // 2, pair, 0)
        drain_add(1, sema)
        drain_in(0, semi)

    pl.run_scoped(scatter_phase,
                  pltpu.SemaphoreType.DMA, pltpu.SemaphoreType.DMA)
    plsc.subcore_barrier()
    for comp, sh in enumerate(shs):
        pltpu.sync_copy(sh.at[pl.ds(sid * NPT, NPT)], zb_v)
        pltpu.sync_copy(
            zb_v, out_hbm.at[pl.ds(cid * 3 * NP + comp * NP + sid * NPT, NPT)])


def _sc_scatter(ti, vx, vy, vz):
    mesh = plsc.VectorSubcoreMesh(core_axis_name="c", subcore_axis_name="s")
    return pl.kernel(
        _scatter_body,
        out_type=jax.ShapeDtypeStruct((NC * 3 * NP,), jnp.float32),
        mesh=mesh,
        scratch_types=[
            pltpu.VMEM((2, CB), jnp.int32),
            pltpu.VMEM((2, 3, CB), jnp.float32),
            pltpu.VMEM((NPT,), jnp.float32),
            pltpu.VMEM_SHARED((NP,), jnp.float32),
            pltpu.VMEM_SHARED((NP,), jnp.float32),
            pltpu.VMEM_SHARED((NP,), jnp.float32),
        ],
        compiler_params=pltpu.CompilerParams(use_tc_tiling_on_sc=False,
                                             needs_layout_passes=False),
    )(ti, vx, vy, vz)


def kernel(rel_selected, target_indices, source_indices, force, viscosity,
           W1, b1, W2, b2, median, contact_distance):
    ti = target_indices.astype(jnp.int32)
    si = source_indices.astype(jnp.int32)
    n = force.shape[0]
    fp = jnp.concatenate([force, jnp.zeros((n, DPAD - 3), jnp.float32)], axis=1)
    relx = rel_selected[:, 0]
    rely = rel_selected[:, 1]
    relz = rel_selected[:, 2]
    w1t = jnp.concatenate([W1[:13], jnp.zeros((3, W1.shape[1]), jnp.float32)],
                          axis=0).T
    b1c = (b1 + viscosity * W1[13])[:, None]
    w2t = W2.T
    b2c = b2[:, None]
    prm = jnp.stack([median, contact_distance]).reshape(1, 2)

    gt = _sc_gather(fp, ti, si)
    vx, vy, vz = _tc_mlp((relx, rely, relz) + tuple(gt), w1t, b1c, w2t, b2c, prm)
    parts = _sc_scatter(ti, vx, vy, vz).reshape(NC, 3, NP)
    return (parts[0] + parts[1])[:, :n].T


# speedup vs baseline: 60.0784x; 1.0137x over previous
import functools

import jax
import jax.numpy as jnp
from jax import lax
from jax.experimental import pallas as pl
from jax.experimental.pallas import tpu as pltpu
from jax.experimental.pallas import tpu_sc as plsc

N_NODES = 100000
N_EDGES = 3200000
NC = 2
NS = 16
NW = NC * NS
LANES = 16

PW = N_EDGES // NW
CB = 2000
NCHUNK = PW // CB
GCB = 800
GNCH = PW // GCB
NB = 5
NP = 100096
NPT = NP // NS

BE = 128000
DPAD = 8


def _gather_body(fp_hbm, ti_hbm, si_hbm,
                 ftx_o, fty_o, ftz_o, fsx_o, fsy_o, fsz_o,
                 ti_v, si_v, trows_v, srows_v, couts_v,
                 semi, semg, semo):
    cid = lax.axis_index("c")
    sid = lax.axis_index("s")
    wid = sid * NC + cid
    dsts = (ftx_o, fty_o, ftz_o, fsx_o, fsy_o, fsz_o)

    def cbase(c):
        return wid * PW + jnp.minimum(c, GNCH - 1) * GCB

    def issue_idx(c, b):
        base = cbase(c)
        pltpu.async_copy(ti_hbm.at[pl.ds(base, GCB)], ti_v.at[b], semi.at[b])
        pltpu.async_copy(si_hbm.at[pl.ds(base, GCB)], si_v.at[b], semi.at[b])

    def drain_idx(b):
        pltpu.make_async_copy(
            ti_hbm.at[pl.ds(0, GCB)], ti_v.at[b], semi.at[b]).wait()
        pltpu.make_async_copy(
            si_hbm.at[pl.ds(0, GCB)], si_v.at[b], semi.at[b]).wait()

    def issue_gather(b):
        pltpu.async_copy(fp_hbm.at[ti_v.at[b]], trows_v.at[b], semg.at[b])
        pltpu.async_copy(fp_hbm.at[si_v.at[b]], srows_v.at[b], semg.at[b])

    def drain_gather(b):
        pltpu.make_async_copy(
            fp_hbm.at[ti_v.at[b]], trows_v.at[b], semg.at[b]).wait()
        pltpu.make_async_copy(
            fp_hbm.at[si_v.at[b]], srows_v.at[b], semg.at[b]).wait()

    def deint(b):
        @plsc.parallel_loop(0, GCB // LANES, step=1, unroll=5)
        def step(i):
            rows = jnp.arange(LANES, dtype=jnp.int32) + i * LANES
            for comp in range(3):
                col = jnp.full((LANES,), comp, dtype=jnp.int32)
                couts_v[b, comp, pl.ds(i * LANES, LANES)] = plsc.load_gather(
                    trows_v.at[b], [rows, col])
                couts_v[b, 3 + comp, pl.ds(i * LANES, LANES)] = plsc.load_gather(
                    srows_v.at[b], [rows, col])

    def issue_out(c, b):
        base = cbase(c)
        for comp in range(6):
            pltpu.async_copy(couts_v.at[b, comp],
                             dsts[comp].at[pl.ds(base, GCB)], semo.at[b])

    def drain_out(b):
        for comp in range(6):
            pltpu.make_async_copy(couts_v.at[b, comp],
                                  dsts[comp].at[pl.ds(0, GCB)],
                                  semo.at[b]).wait()

    def body(c, b, first):
        drain_gather(b)
        if not first:
            drain_out(b)
        deint(b)
        issue_out(c, b)
        b2 = (b + 2) % NB
        drain_idx(b2)
        issue_gather(b2)
        issue_idx(c + 4, (b + 4) % NB)

    for c0 in range(4):
        issue_idx(c0, c0)
    drain_idx(0)
    issue_gather(0)
    drain_idx(1)
    issue_gather(1)
    for c0 in range(NB):
        body(c0, c0, True)

    def group(j, _):
        for b0 in range(NB):
            body(NB * j + b0, b0, False)
        return 0

    lax.fori_loop(1, GNCH // NB, group, 0)
    drain_gather(0)
    drain_gather(1)
    drain_idx(2)
    drain_idx(3)
    for b0 in range(NB):
        drain_out(b0)


def _sc_gather(fp, ti, si):
    e = jax.ShapeDtypeStruct((N_EDGES,), jnp.float32)
    mesh = plsc.VectorSubcoreMesh(core_axis_name="c", subcore_axis_name="s")
    return pl.kernel(
        _gather_body,
        out_type=(e,) * 6,
        mesh=mesh,
        scratch_types=[
            pltpu.VMEM((NB, GCB), jnp.int32),
            pltpu.VMEM((NB, GCB), jnp.int32),
            pltpu.VMEM((NB, GCB, DPAD), jnp.float32),
            pltpu.VMEM((NB, GCB, DPAD), jnp.float32),
            pltpu.VMEM((NB, 6, GCB), jnp.float32),
            pltpu.SemaphoreType.DMA((NB,)),
            pltpu.SemaphoreType.DMA((NB,)),
            pltpu.SemaphoreType.DMA((NB,)),
        ],
        compiler_params=pltpu.CompilerParams(use_tc_tiling_on_sc=False,
                                             needs_layout_passes=False),
    )(fp, ti, si)


def _mlp_body(rx, ry, rz, gtx, gty, gtz, gsx, gsy, gsz,
              w1t, b1c, w2t, b2c, prm, ox, oy, oz):
    def row(r):
        return r[...].reshape(1, BE)

    x, y, z = row(rx), row(ry), row(rz)
    d = jnp.sqrt(x * x + y * y + z * z)
    d = jnp.maximum(d, 1e-8)
    m = prm[0:1, 0:1]
    cd = prm[0:1, 1:2]
    rs = d - m
    rsq = rs * rs
    rq = rsq * rsq
    mind = d - cd
    feats = jnp.concatenate(
        [x, y, z, d, rsq, rq, mind,
         row(gtx), row(gty), row(gtz), row(gsx), row(gsy), row(gsz),
         jnp.zeros((3, BE), jnp.float32)], axis=0)
    h = jnp.tanh(
        jnp.dot(w1t[...], feats, preferred_element_type=jnp.float32) + b1c[...])
    v = jnp.dot(w2t[...], h, preferred_element_type=jnp.float32) + b2c[...]
    ox[...] = v[0:1, :].reshape(BE)
    oy[...] = v[1:2, :].reshape(BE)
    oz[...] = v[2:3, :].reshape(BE)


def _tc_mlp(comps, w1t, b1c, w2t, b2c, prm):
    grid = (N_EDGES // BE,)
    espec = pl.BlockSpec((BE,), lambda i: (i,))
    wspec = lambda shape: pl.BlockSpec(shape, lambda i: tuple(0 for _ in shape))
    e = jax.ShapeDtypeStruct((N_EDGES,), jnp.float32)
    return pl.pallas_call(
        _mlp_body,
        grid=grid,
        in_specs=[espec] * 9 + [wspec((32, 16)), wspec((32, 1)),
                                wspec((3, 32)), wspec((3, 1)), wspec((1, 2))],
        out_specs=[espec] * 3,
        out_shape=(e, e, e),
    )(*comps, w1t, b1c, w2t, b2c, prm)


def _scatter_body(ti_hbm, vx_hbm, vy_hbm, vz_hbm, out_hbm,
                  ti_v, vv_v, zb_v, shx, shy, shz):
    cid = lax.axis_index("c")
    sid = lax.axis_index("s")
    wid = sid * NC + cid
    shs = (shx, shy, shz)
    vins = (vx_hbm, vy_hbm, vz_hbm)

    def zb(i, _):
        zb_v[pl.ds(i * LANES, LANES)] = jnp.zeros((LANES,), jnp.float32)
        return 0

    lax.fori_loop(0, NPT // LANES, zb, 0)
    for sh in shs:
        pltpu.sync_copy(zb_v, sh.at[pl.ds(sid * NPT, NPT)])
    plsc.subcore_barrier()

    def cbase(c):
        return wid * PW + jnp.minimum(c, NCHUNK - 1) * CB

    def issue_in(c, b, semi):
        base = cbase(c)
        pltpu.async_copy(ti_hbm.at[pl.ds(base, CB)], ti_v.at[b], semi)
        for comp in range(3):
            pltpu.async_copy(vins[comp].at[pl.ds(base, CB)],
                             vv_v.at[b, comp], semi)

    def drain_in(b, semi):
        pltpu.make_async_copy(ti_hbm.at[pl.ds(0, CB)], ti_v.at[b], semi).wait()
        for comp in range(3):
            pltpu.make_async_copy(vins[comp].at[pl.ds(0, CB)],
                                  vv_v.at[b, comp], semi).wait()

    def issue_add(b, sema):
        for comp in range(3):
            pltpu.async_copy(vv_v.at[b, comp], shs[comp].at[ti_v.at[b]],
                             sema, add=True)

    def drain_add(b, sema):
        for comp in range(3):
            pltpu.make_async_copy(vv_v.at[b, comp],
                                  shs[comp].at[ti_v.at[b]], sema).wait()

    def run_chunk(c, b, first, semi, sema):
        drain_in(b, semi)
        if not first:
            drain_add(1 - b, sema)
        issue_in(c + 1, 1 - b, semi)
        issue_add(b, sema)

    def scatter_phase(semi, sema):
        issue_in(0, 0, semi)
        run_chunk(0, 0, True, semi, sema)
        run_chunk(1, 1, False, semi, sema)

        def pair(j, _):
            run_chunk(2 * j, 0, False, semi, sema)
            run_chunk(2 * j + 1, 1, False, semi, sema)
            return 0

        lax.fori_loop(1, NCHUNK // 2, pair, 0)
        drain_add(1, sema)
        drain_in(0, semi)

    pl.run_scoped(scatter_phase,
                  pltpu.SemaphoreType.DMA, pltpu.SemaphoreType.DMA)
    plsc.subcore_barrier()
    for comp, sh in enumerate(shs):
        pltpu.sync_copy(sh.at[pl.ds(sid * NPT, NPT)], zb_v)
        pltpu.sync_copy(
            zb_v, out_hbm.at[pl.ds(cid * 3 * NP + comp * NP + sid * NPT, NPT)])


def _sc_scatter(ti, vx, vy, vz):
    mesh = plsc.VectorSubcoreMesh(core_axis_name="c", subcore_axis_name="s")
    return pl.kernel(
        _scatter_body,
        out_type=jax.ShapeDtypeStruct((NC * 3 * NP,), jnp.float32),
        mesh=mesh,
        scratch_types=[
            pltpu.VMEM((2, CB), jnp.int32),
            pltpu.VMEM((2, 3, CB), jnp.float32),
            pltpu.VMEM((NPT,), jnp.float32),
            pltpu.VMEM_SHARED((NP,), jnp.float32),
            pltpu.VMEM_SHARED((NP,), jnp.float32),
            pltpu.VMEM_SHARED((NP,), jnp.float32),
        ],
        compiler_params=pltpu.CompilerParams(use_tc_tiling_on_sc=False,
                                             needs_layout_passes=False),
    )(ti, vx, vy, vz)


def kernel(rel_selected, target_indices, source_indices, force, viscosity,
           W1, b1, W2, b2, median, contact_distance):
    ti = target_indices.astype(jnp.int32)
    si = source_indices.astype(jnp.int32)
    n = force.shape[0]
    fp = jnp.concatenate([force, jnp.zeros((n, DPAD - 3), jnp.float32)], axis=1)
    relx = rel_selected[:, 0]
    rely = rel_selected[:, 1]
    relz = rel_selected[:, 2]
    w1t = jnp.concatenate([W1[:13], jnp.zeros((3, W1.shape[1]), jnp.float32)],
                          axis=0).T
    b1c = (b1 + viscosity * W1[13])[:, None]
    w2t = W2.T
    b2c = b2[:, None]
    prm = jnp.stack([median, contact_distance]).reshape(1, 2)

    gt = _sc_gather(fp, ti, si)
    vx, vy, vz = _tc_mlp((relx, rely, relz) + tuple(gt), w1t, b1c, w2t, b2c, prm)
    parts = _sc_scatter(ti, vx, vy, vz).reshape(NC, 3, NP)
    return (parts[0] + parts[1])[:, :n].T


# speedup vs baseline: 60.1108x; 1.0005x over previous
import functools

import jax
import jax.numpy as jnp
from jax import lax
from jax.experimental import pallas as pl
from jax.experimental.pallas import tpu as pltpu
from jax.experimental.pallas import tpu_sc as plsc

N_NODES = 100000
N_EDGES = 3200000
NC = 2
NS = 16
NW = NC * NS
LANES = 16

PW = N_EDGES // NW
CB = 5000
NCHUNK = PW // CB
GCB = 800
GNCH = PW // GCB
NB = 5
NP = 100096
NPT = NP // NS

BE = 128000
DPAD = 8


def _gather_body(fp_hbm, ti_hbm, si_hbm,
                 ftx_o, fty_o, ftz_o, fsx_o, fsy_o, fsz_o,
                 ti_v, si_v, trows_v, srows_v, couts_v,
                 semi, semg, semo):
    cid = lax.axis_index("c")
    sid = lax.axis_index("s")
    wid = sid * NC + cid
    dsts = (ftx_o, fty_o, ftz_o, fsx_o, fsy_o, fsz_o)

    def cbase(c):
        return wid * PW + jnp.minimum(c, GNCH - 1) * GCB

    def issue_idx(c, b):
        base = cbase(c)
        pltpu.async_copy(ti_hbm.at[pl.ds(base, GCB)], ti_v.at[b], semi.at[b])
        pltpu.async_copy(si_hbm.at[pl.ds(base, GCB)], si_v.at[b], semi.at[b])

    def drain_idx(b):
        pltpu.make_async_copy(
            ti_hbm.at[pl.ds(0, GCB)], ti_v.at[b], semi.at[b]).wait()
        pltpu.make_async_copy(
            si_hbm.at[pl.ds(0, GCB)], si_v.at[b], semi.at[b]).wait()

    def issue_gather(b):
        pltpu.async_copy(fp_hbm.at[ti_v.at[b]], trows_v.at[b], semg.at[b])
        pltpu.async_copy(fp_hbm.at[si_v.at[b]], srows_v.at[b], semg.at[b])

    def drain_gather(b):
        pltpu.make_async_copy(
            fp_hbm.at[ti_v.at[b]], trows_v.at[b], semg.at[b]).wait()
        pltpu.make_async_copy(
            fp_hbm.at[si_v.at[b]], srows_v.at[b], semg.at[b]).wait()

    def deint(b):
        @plsc.parallel_loop(0, GCB // LANES, step=1, unroll=5)
        def step(i):
            rows = jnp.arange(LANES, dtype=jnp.int32) + i * LANES
            for comp in range(3):
                col = jnp.full((LANES,), comp, dtype=jnp.int32)
                couts_v[b, comp, pl.ds(i * LANES, LANES)] = plsc.load_gather(
                    trows_v.at[b], [rows, col])
                couts_v[b, 3 + comp, pl.ds(i * LANES, LANES)] = plsc.load_gather(
                    srows_v.at[b], [rows, col])

    def issue_out(c, b):
        base = cbase(c)
        for comp in range(6):
            pltpu.async_copy(couts_v.at[b, comp],
                             dsts[comp].at[pl.ds(base, GCB)], semo.at[b])

    def drain_out(b):
        for comp in range(6):
            pltpu.make_async_copy(couts_v.at[b, comp],
                                  dsts[comp].at[pl.ds(0, GCB)],
                                  semo.at[b]).wait()

    def body(c, b, first):
        drain_gather(b)
        if not first:
            drain_out(b)
        deint(b)
        issue_out(c, b)
        b2 = (b + 2) % NB
        drain_idx(b2)
        issue_gather(b2)
        issue_idx(c + 4, (b + 4) % NB)

    for c0 in range(4):
        issue_idx(c0, c0)
    drain_idx(0)
    issue_gather(0)
    drain_idx(1)
    issue_gather(1)
    for c0 in range(NB):
        body(c0, c0, True)

    def group(j, _):
        for b0 in range(NB):
            body(NB * j + b0, b0, False)
        return 0

    lax.fori_loop(1, GNCH // NB, group, 0)
    drain_gather(0)
    drain_gather(1)
    drain_idx(2)
    drain_idx(3)
    for b0 in range(NB):
        drain_out(b0)


def _sc_gather(fp, ti, si):
    e = jax.ShapeDtypeStruct((N_EDGES,), jnp.float32)
    mesh = plsc.VectorSubcoreMesh(core_axis_name="c", subcore_axis_name="s")
    return pl.kernel(
        _gather_body,
        out_type=(e,) * 6,
        mesh=mesh,
        scratch_types=[
            pltpu.VMEM((NB, GCB), jnp.int32),
            pltpu.VMEM((NB, GCB), jnp.int32),
            pltpu.VMEM((NB, GCB, DPAD), jnp.float32),
            pltpu.VMEM((NB, GCB, DPAD), jnp.float32),
            pltpu.VMEM((NB, 6, GCB), jnp.float32),
            pltpu.SemaphoreType.DMA((NB,)),
            pltpu.SemaphoreType.DMA((NB,)),
            pltpu.SemaphoreType.DMA((NB,)),
        ],
        compiler_params=pltpu.CompilerParams(use_tc_tiling_on_sc=False,
                                             needs_layout_passes=False),
    )(fp, ti, si)


def _mlp_body(rx, ry, rz, gtx, gty, gtz, gsx, gsy, gsz,
              w1t, b1c, w2t, b2c, prm, ox, oy, oz):
    def row(r):
        return r[...].reshape(1, BE)

    x, y, z = row(rx), row(ry), row(rz)
    d = jnp.sqrt(x * x + y * y + z * z)
    d = jnp.maximum(d, 1e-8)
    m = prm[0:1, 0:1]
    cd = prm[0:1, 1:2]
    rs = d - m
    rsq = rs * rs
    rq = rsq * rsq
    mind = d - cd
    feats = jnp.concatenate(
        [x, y, z, d, rsq, rq, mind,
         row(gtx), row(gty), row(gtz), row(gsx), row(gsy), row(gsz),
         jnp.zeros((3, BE), jnp.float32)], axis=0)
    h = jnp.tanh(
        jnp.dot(w1t[...], feats, preferred_element_type=jnp.float32) + b1c[...])
    v = jnp.dot(w2t[...], h, preferred_element_type=jnp.float32) + b2c[...]
    ox[...] = v[0:1, :].reshape(BE)
    oy[...] = v[1:2, :].reshape(BE)
    oz[...] = v[2:3, :].reshape(BE)


def _tc_mlp(comps, w1t, b1c, w2t, b2c, prm):
    grid = (N_EDGES // BE,)
    espec = pl.BlockSpec((BE,), lambda i: (i,))
    wspec = lambda shape: pl.BlockSpec(shape, lambda i: tuple(0 for _ in shape))
    e = jax.ShapeDtypeStruct((N_EDGES,), jnp.float32)
    return pl.pallas_call(
        _mlp_body,
        grid=grid,
        in_specs=[espec] * 9 + [wspec((32, 16)), wspec((32, 1)),
                                wspec((3, 32)), wspec((3, 1)), wspec((1, 2))],
        out_specs=[espec] * 3,
        out_shape=(e, e, e),
    )(*comps, w1t, b1c, w2t, b2c, prm)


def _scatter_body(ti_hbm, vx_hbm, vy_hbm, vz_hbm, out_hbm,
                  ti_v, vv_v, zb_v, shx, shy, shz):
    cid = lax.axis_index("c")
    sid = lax.axis_index("s")
    wid = sid * NC + cid
    shs = (shx, shy, shz)
    vins = (vx_hbm, vy_hbm, vz_hbm)

    def zb(i, _):
        zb_v[pl.ds(i * LANES, LANES)] = jnp.zeros((LANES,), jnp.float32)
        return 0

    lax.fori_loop(0, NPT // LANES, zb, 0)
    for sh in shs:
        pltpu.sync_copy(zb_v, sh.at[pl.ds(sid * NPT, NPT)])
    plsc.subcore_barrier()

    def cbase(c):
        return wid * PW + jnp.minimum(c, NCHUNK - 1) * CB

    def issue_in(c, b, semi):
        base = cbase(c)
        pltpu.async_copy(ti_hbm.at[pl.ds(base, CB)], ti_v.at[b], semi)
        for comp in range(3):
            pltpu.async_copy(vins[comp].at[pl.ds(base, CB)],
                             vv_v.at[b, comp], semi)

    def drain_in(b, semi):
        pltpu.make_async_copy(ti_hbm.at[pl.ds(0, CB)], ti_v.at[b], semi).wait()
        for comp in range(3):
            pltpu.make_async_copy(vins[comp].at[pl.ds(0, CB)],
                                  vv_v.at[b, comp], semi).wait()

    def issue_add(b, sema):
        for comp in range(3):
            pltpu.async_copy(vv_v.at[b, comp], shs[comp].at[ti_v.at[b]],
                             sema, add=True)

    def drain_add(b, sema):
        for comp in range(3):
            pltpu.make_async_copy(vv_v.at[b, comp],
                                  shs[comp].at[ti_v.at[b]], sema).wait()

    def run_chunk(c, b, first, semi, sema):
        drain_in(b, semi)
        if not first:
            drain_add(1 - b, sema)
        issue_in(c + 1, 1 - b, semi)
        issue_add(b, sema)

    def scatter_phase(semi, sema):
        issue_in(0, 0, semi)
        run_chunk(0, 0, True, semi, sema)
        run_chunk(1, 1, False, semi, sema)

        def pair(j, _):
            run_chunk(2 * j, 0, False, semi, sema)
            run_chunk(2 * j + 1, 1, False, semi, sema)
            return 0

        lax.fori_loop(1, NCHUNK // 2, pair, 0)
        drain_add(1, sema)
        drain_in(0, semi)

    pl.run_scoped(scatter_phase,
                  pltpu.SemaphoreType.DMA, pltpu.SemaphoreType.DMA)
    plsc.subcore_barrier()
    for comp, sh in enumerate(shs):
        pltpu.sync_copy(sh.at[pl.ds(sid * NPT, NPT)], zb_v)
        pltpu.sync_copy(
            zb_v, out_hbm.at[pl.ds(cid * 3 * NP + comp * NP + sid * NPT, NPT)])


def _sc_scatter(ti, vx, vy, vz):
    mesh = plsc.VectorSubcoreMesh(core_axis_name="c", subcore_axis_name="s")
    return pl.kernel(
        _scatter_body,
        out_type=jax.ShapeDtypeStruct((NC * 3 * NP,), jnp.float32),
        mesh=mesh,
        scratch_types=[
            pltpu.VMEM((2, CB), jnp.int32),
            pltpu.VMEM((2, 3, CB), jnp.float32),
            pltpu.VMEM((NPT,), jnp.float32),
            pltpu.VMEM_SHARED((NP,), jnp.float32),
            pltpu.VMEM_SHARED((NP,), jnp.float32),
            pltpu.VMEM_SHARED((NP,), jnp.float32),
        ],
        compiler_params=pltpu.CompilerParams(use_tc_tiling_on_sc=False,
                                             needs_layout_passes=False),
    )(ti, vx, vy, vz)


def kernel(rel_selected, target_indices, source_indices, force, viscosity,
           W1, b1, W2, b2, median, contact_distance):
    ti = target_indices.astype(jnp.int32)
    si = source_indices.astype(jnp.int32)
    n = force.shape[0]
    fp = jnp.concatenate([force, jnp.zeros((n, DPAD - 3), jnp.float32)], axis=1)
    relx = rel_selected[:, 0]
    rely = rel_selected[:, 1]
    relz = rel_selected[:, 2]
    w1t = jnp.concatenate([W1[:13], jnp.zeros((3, W1.shape[1]), jnp.float32)],
                          axis=0).T
    b1c = (b1 + viscosity * W1[13])[:, None]
    w2t = W2.T
    b2c = b2[:, None]
    prm = jnp.stack([median, contact_distance]).reshape(1, 2)

    gt = _sc_gather(fp, ti, si)
    vx, vy, vz = _tc_mlp((relx, rely, relz) + tuple(gt), w1t, b1c, w2t, b2c, prm)
    parts = _sc_scatter(ti, vx, vy, vz).reshape(NC, 3, NP)
    return (parts[0] + parts[1])[:, :n].T


# speedup vs baseline: 60.4619x; 1.0058x over previous
import functools

import jax
import jax.numpy as jnp
from jax import lax
from jax.experimental import pallas as pl
from jax.experimental.pallas import tpu as pltpu
from jax.experimental.pallas import tpu_sc as plsc

N_NODES = 100000
N_EDGES = 3200000
NC = 2
NS = 16
NW = NC * NS
LANES = 16

PW = N_EDGES // NW
CB = 5000
NCHUNK = PW // CB
GCB = 800
GNCH = PW // GCB
NB = 5
NP = 100096
NPT = NP // NS

BE = 128000
DPAD = 8


def _gather_body(fp_hbm, ti_hbm, si_hbm,
                 ftx_o, fty_o, ftz_o, fsx_o, fsy_o, fsz_o,
                 ti_v, si_v, trows_v, srows_v, couts_v,
                 semi, semg, semo):
    cid = lax.axis_index("c")
    sid = lax.axis_index("s")
    wid = sid * NC + cid
    dsts = (ftx_o, fty_o, ftz_o, fsx_o, fsy_o, fsz_o)

    def cbase(c):
        return wid * PW + jnp.minimum(c, GNCH - 1) * GCB

    def issue_idx(c, b):
        base = cbase(c)
        pltpu.async_copy(ti_hbm.at[pl.ds(base, GCB)], ti_v.at[b], semi.at[b])
        pltpu.async_copy(si_hbm.at[pl.ds(base, GCB)], si_v.at[b], semi.at[b])

    def drain_idx(b):
        pltpu.make_async_copy(
            ti_hbm.at[pl.ds(0, GCB)], ti_v.at[b], semi.at[b]).wait()
        pltpu.make_async_copy(
            si_hbm.at[pl.ds(0, GCB)], si_v.at[b], semi.at[b]).wait()

    def issue_gather(b):
        pltpu.async_copy(fp_hbm.at[ti_v.at[b]], trows_v.at[b], semg.at[b])
        pltpu.async_copy(fp_hbm.at[si_v.at[b]], srows_v.at[b], semg.at[b])

    def drain_gather(b):
        pltpu.make_async_copy(
            fp_hbm.at[ti_v.at[b]], trows_v.at[b], semg.at[b]).wait()
        pltpu.make_async_copy(
            fp_hbm.at[si_v.at[b]], srows_v.at[b], semg.at[b]).wait()

    def deint(b):
        @plsc.parallel_loop(0, GCB // LANES, step=1, unroll=5)
        def step(i):
            rows = jnp.arange(LANES, dtype=jnp.int32) + i * LANES
            for comp in range(3):
                col = jnp.full((LANES,), comp, dtype=jnp.int32)
                couts_v[b, comp, pl.ds(i * LANES, LANES)] = plsc.load_gather(
                    trows_v.at[b], [rows, col])
                couts_v[b, 3 + comp, pl.ds(i * LANES, LANES)] = plsc.load_gather(
                    srows_v.at[b], [rows, col])

    def issue_out(c, b):
        base = cbase(c)
        for comp in range(6):
            pltpu.async_copy(couts_v.at[b, comp],
                             dsts[comp].at[pl.ds(base, GCB)], semo.at[b])

    def drain_out(b):
        for comp in range(6):
            pltpu.make_async_copy(couts_v.at[b, comp],
                                  dsts[comp].at[pl.ds(0, GCB)],
                                  semo.at[b]).wait()

    def body(c, b, first):
        drain_gather(b)
        if not first:
            drain_out(b)
        deint(b)
        issue_out(c, b)
        b3 = (b + 3) % NB
        drain_idx(b3)
        issue_gather(b3)
        issue_idx(c + 4, (b + 4) % NB)

    for c0 in range(4):
        issue_idx(c0, c0)
    for c0 in range(3):
        drain_idx(c0)
        issue_gather(c0)
    for c0 in range(NB):
        body(c0, c0, True)

    def group(j, _):
        for b0 in range(NB):
            body(NB * j + b0, b0, False)
        return 0

    lax.fori_loop(1, GNCH // NB, group, 0)
    drain_gather(0)
    drain_gather(1)
    drain_gather(2)
    drain_idx(3)
    for b0 in range(NB):
        drain_out(b0)


def _sc_gather(fp, ti, si):
    e = jax.ShapeDtypeStruct((N_EDGES,), jnp.float32)
    mesh = plsc.VectorSubcoreMesh(core_axis_name="c", subcore_axis_name="s")
    return pl.kernel(
        _gather_body,
        out_type=(e,) * 6,
        mesh=mesh,
        scratch_types=[
            pltpu.VMEM((NB, GCB), jnp.int32),
            pltpu.VMEM((NB, GCB), jnp.int32),
            pltpu.VMEM((NB, GCB, DPAD), jnp.float32),
            pltpu.VMEM((NB, GCB, DPAD), jnp.float32),
            pltpu.VMEM((NB, 6, GCB), jnp.float32),
            pltpu.SemaphoreType.DMA((NB,)),
            pltpu.SemaphoreType.DMA((NB,)),
            pltpu.SemaphoreType.DMA((NB,)),
        ],
        compiler_params=pltpu.CompilerParams(use_tc_tiling_on_sc=False,
                                             needs_layout_passes=False),
    )(fp, ti, si)


def _mlp_body(rx, ry, rz, gtx, gty, gtz, gsx, gsy, gsz,
              w1t, b1c, w2t, b2c, prm, ox, oy, oz):
    def row(r):
        return r[...].reshape(1, BE)

    x, y, z = row(rx), row(ry), row(rz)
    d = jnp.sqrt(x * x + y * y + z * z)
    d = jnp.maximum(d, 1e-8)
    m = prm[0:1, 0:1]
    cd = prm[0:1, 1:2]
    rs = d - m
    rsq = rs * rs
    rq = rsq * rsq
    mind = d - cd
    feats = jnp.concatenate(
        [x, y, z, d, rsq, rq, mind,
         row(gtx), row(gty), row(gtz), row(gsx), row(gsy), row(gsz),
         jnp.zeros((3, BE), jnp.float32)], axis=0)
    h = jnp.tanh(
        jnp.dot(w1t[...], feats, preferred_element_type=jnp.float32) + b1c[...])
    v = jnp.dot(w2t[...], h, preferred_element_type=jnp.float32) + b2c[...]
    ox[...] = v[0:1, :].reshape(BE)
    oy[...] = v[1:2, :].reshape(BE)
    oz[...] = v[2:3, :].reshape(BE)


def _tc_mlp(comps, w1t, b1c, w2t, b2c, prm):
    grid = (N_EDGES // BE,)
    espec = pl.BlockSpec((BE,), lambda i: (i,))
    wspec = lambda shape: pl.BlockSpec(shape, lambda i: tuple(0 for _ in shape))
    e = jax.ShapeDtypeStruct((N_EDGES,), jnp.float32)
    return pl.pallas_call(
        _mlp_body,
        grid=grid,
        in_specs=[espec] * 9 + [wspec((32, 16)), wspec((32, 1)),
                                wspec((3, 32)), wspec((3, 1)), wspec((1, 2))],
        out_specs=[espec] * 3,
        out_shape=(e, e, e),
    )(*comps, w1t, b1c, w2t, b2c, prm)


def _scatter_body(ti_hbm, vx_hbm, vy_hbm, vz_hbm, out_hbm,
                  ti_v, vv_v, zb_v, shx, shy, shz):
    cid = lax.axis_index("c")
    sid = lax.axis_index("s")
    wid = sid * NC + cid
    shs = (shx, shy, shz)
    vins = (vx_hbm, vy_hbm, vz_hbm)

    def zb(i, _):
        zb_v[pl.ds(i * LANES, LANES)] = jnp.zeros((LANES,), jnp.float32)
        return 0

    lax.fori_loop(0, NPT // LANES, zb, 0)
    for sh in shs:
        pltpu.sync_copy(zb_v, sh.at[pl.ds(sid * NPT, NPT)])
    plsc.subcore_barrier()

    def cbase(c):
        return wid * PW + jnp.minimum(c, NCHUNK - 1) * CB

    def issue_in(c, b, semi):
        base = cbase(c)
        pltpu.async_copy(ti_hbm.at[pl.ds(base, CB)], ti_v.at[b], semi)
        for comp in range(3):
            pltpu.async_copy(vins[comp].at[pl.ds(base, CB)],
                             vv_v.at[b, comp], semi)

    def drain_in(b, semi):
        pltpu.make_async_copy(ti_hbm.at[pl.ds(0, CB)], ti_v.at[b], semi).wait()
        for comp in range(3):
            pltpu.make_async_copy(vins[comp].at[pl.ds(0, CB)],
                                  vv_v.at[b, comp], semi).wait()

    def issue_add(b, sema):
        for comp in range(3):
            pltpu.async_copy(vv_v.at[b, comp], shs[comp].at[ti_v.at[b]],
                             sema, add=True)

    def drain_add(b, sema):
        for comp in range(3):
            pltpu.make_async_copy(vv_v.at[b, comp],
                                  shs[comp].at[ti_v.at[b]], sema).wait()

    def run_chunk(c, b, first, semi, sema):
        drain_in(b, semi)
        if not first:
            drain_add(1 - b, sema)
        issue_in(c + 1, 1 - b, semi)
        issue_add(b, sema)

    def scatter_phase(semi, sema):
        issue_in(0, 0, semi)
        run_chunk(0, 0, True, semi, sema)
        run_chunk(1, 1, False, semi, sema)

        def pair(j, _):
            run_chunk(2 * j, 0, False, semi, sema)
            run_chunk(2 * j + 1, 1, False, semi, sema)
            return 0

        lax.fori_loop(1, NCHUNK // 2, pair, 0)
        drain_add(1, sema)
        drain_in(0, semi)

    pl.run_scoped(scatter_phase,
                  pltpu.SemaphoreType.DMA, pltpu.SemaphoreType.DMA)
    plsc.subcore_barrier()
    for comp, sh in enumerate(shs):
        pltpu.sync_copy(sh.at[pl.ds(sid * NPT, NPT)], zb_v)
        pltpu.sync_copy(
            zb_v, out_hbm.at[pl.ds(cid * 3 * NP + comp * NP + sid * NPT, NPT)])


def _sc_scatter(ti, vx, vy, vz):
    mesh = plsc.VectorSubcoreMesh(core_axis_name="c", subcore_axis_name="s")
    return pl.kernel(
        _scatter_body,
        out_type=jax.ShapeDtypeStruct((NC * 3 * NP,), jnp.float32),
        mesh=mesh,
        scratch_types=[
            pltpu.VMEM((2, CB), jnp.int32),
            pltpu.VMEM((2, 3, CB), jnp.float32),
            pltpu.VMEM((NPT,), jnp.float32),
            pltpu.VMEM_SHARED((NP,), jnp.float32),
            pltpu.VMEM_SHARED((NP,), jnp.float32),
            pltpu.VMEM_SHARED((NP,), jnp.float32),
        ],
        compiler_params=pltpu.CompilerParams(use_tc_tiling_on_sc=False,
                                             needs_layout_passes=False),
    )(ti, vx, vy, vz)


def kernel(rel_selected, target_indices, source_indices, force, viscosity,
           W1, b1, W2, b2, median, contact_distance):
    ti = target_indices.astype(jnp.int32)
    si = source_indices.astype(jnp.int32)
    n = force.shape[0]
    fp = jnp.concatenate([force, jnp.zeros((n, DPAD - 3), jnp.float32)], axis=1)
    relx = rel_selected[:, 0]
    rely = rel_selected[:, 1]
    relz = rel_selected[:, 2]
    w1t = jnp.concatenate([W1[:13], jnp.zeros((3, W1.shape[1]), jnp.float32)],
                          axis=0).T
    b1c = (b1 + viscosity * W1[13])[:, None]
    w2t = W2.T
    b2c = b2[:, None]
    prm = jnp.stack([median, contact_distance]).reshape(1, 2)

    gt = _sc_gather(fp, ti, si)
    vx, vy, vz = _tc_mlp((relx, rely, relz) + tuple(gt), w1t, b1c, w2t, b2c, prm)
    parts = _sc_scatter(ti, vx, vy, vz).reshape(NC, 3, NP)
    return (parts[0] + parts[1])[:, :n].T


# speedup vs baseline: 60.5121x; 1.0008x over previous
import jax
import jax.numpy as jnp
from jax import lax
from jax.experimental import pallas as pl
from jax.experimental.pallas import tpu as pltpu
from jax.experimental.pallas import tpu_sc as plsc

N_NODES = 100000
N_EDGES = 3200000
NC = 2
NS = 16
NW = NC * NS
LANES = 16

PW = N_EDGES // NW
CB = 5000
NCHUNK = PW // CB
GCB = 800
GNCH = PW // GCB
NB = 5
NP = 100096
NPT = NP // NS

BE = 128000
DPAD = 8


def _gather_body(fp_hbm, ti_hbm, si_hbm,
                 ftx_o, fty_o, ftz_o, fsx_o, fsy_o, fsz_o,
                 ti_v, si_v, trows_v, srows_v, couts_v,
                 semi, semg, semo):
    cid = lax.axis_index("c")
    sid = lax.axis_index("s")
    wid = sid * NC + cid
    dsts = (ftx_o, fty_o, ftz_o, fsx_o, fsy_o, fsz_o)

    def cbase(c):
        return wid * PW + jnp.minimum(c, GNCH - 1) * GCB

    def issue_idx(c, b):
        base = cbase(c)
        pltpu.async_copy(ti_hbm.at[pl.ds(base, GCB)], ti_v.at[b], semi.at[b])
        pltpu.async_copy(si_hbm.at[pl.ds(base, GCB)], si_v.at[b], semi.at[b])

    def drain_idx(b):
        pltpu.make_async_copy(
            ti_hbm.at[pl.ds(0, GCB)], ti_v.at[b], semi.at[b]).wait()
        pltpu.make_async_copy(
            si_hbm.at[pl.ds(0, GCB)], si_v.at[b], semi.at[b]).wait()

    def issue_gather(b):
        pltpu.async_copy(fp_hbm.at[ti_v.at[b]], trows_v.at[b], semg.at[b])
        pltpu.async_copy(fp_hbm.at[si_v.at[b]], srows_v.at[b], semg.at[b])

    def drain_gather(b):
        pltpu.make_async_copy(
            fp_hbm.at[ti_v.at[b]], trows_v.at[b], semg.at[b]).wait()
        pltpu.make_async_copy(
            fp_hbm.at[si_v.at[b]], srows_v.at[b], semg.at[b]).wait()

    def deint(b):
        @plsc.parallel_loop(0, GCB // LANES, step=1, unroll=5)
        def step(i):
            rows = jnp.arange(LANES, dtype=jnp.int32) + i * LANES
            for comp in range(3):
                col = jnp.full((LANES,), comp, dtype=jnp.int32)
                couts_v[b, comp, pl.ds(i * LANES, LANES)] = plsc.load_gather(
                    trows_v.at[b], [rows, col])
                couts_v[b, 3 + comp, pl.ds(i * LANES, LANES)] = plsc.load_gather(
                    srows_v.at[b], [rows, col])

    def issue_out(c, b):
        base = cbase(c)
        for comp in range(6):
            pltpu.async_copy(couts_v.at[b, comp],
                             dsts[comp].at[pl.ds(base, GCB)], semo.at[b])

    def drain_out(b):
        for comp in range(6):
            pltpu.make_async_copy(couts_v.at[b, comp],
                                  dsts[comp].at[pl.ds(0, GCB)],
                                  semo.at[b]).wait()

    def body(c, b, first):
        drain_gather(b)
        if not first:
            drain_out(b)
        deint(b)
        issue_out(c, b)
        b3 = (b + 3) % NB
        drain_idx(b3)
        issue_gather(b3)
        issue_idx(c + 4, (b + 4) % NB)

    for c0 in range(4):
        issue_idx(c0, c0)
    for c0 in range(3):
        drain_idx(c0)
        issue_gather(c0)
    for c0 in range(NB):
        body(c0, c0, True)

    def group(j, _):
        for b0 in range(NB):
            body(NB * j + b0, b0, False)
        return 0

    lax.fori_loop(1, GNCH // NB, group, 0)
    drain_gather(0)
    drain_gather(1)
    drain_gather(2)
    drain_idx(3)
    for b0 in range(NB):
        drain_out(b0)


def _sc_gather(fp, ti, si):
    e = jax.ShapeDtypeStruct((N_EDGES,), jnp.float32)
    mesh = plsc.VectorSubcoreMesh(core_axis_name="c", subcore_axis_name="s")
    return pl.kernel(
        _gather_body,
        out_type=(e,) * 6,
        mesh=mesh,
        scratch_types=[
            pltpu.VMEM((NB, GCB), jnp.int32),
            pltpu.VMEM((NB, GCB), jnp.int32),
            pltpu.VMEM((NB, GCB, DPAD), jnp.float32),
            pltpu.VMEM((NB, GCB, DPAD), jnp.float32),
            pltpu.VMEM((NB, 6, GCB), jnp.float32),
            pltpu.SemaphoreType.DMA((NB,)),
            pltpu.SemaphoreType.DMA((NB,)),
            pltpu.SemaphoreType.DMA((NB,)),
        ],
        compiler_params=pltpu.CompilerParams(use_tc_tiling_on_sc=False,
                                             needs_layout_passes=False),
    )(fp, ti, si)


def _mlp_body(rx, ry, rz, gtx, gty, gtz, gsx, gsy, gsz,
              w1t, b1c, w2t, b2c, prm, ox, oy, oz):
    def row(r):
        return r[...].reshape(1, BE)

    x, y, z = row(rx), row(ry), row(rz)
    d = jnp.sqrt(x * x + y * y + z * z)
    d = jnp.maximum(d, 1e-8)
    m = prm[0:1, 0:1]
    cd = prm[0:1, 1:2]
    rs = d - m
    rsq = rs * rs
    rq = rsq * rsq
    mind = d - cd
    feats = jnp.concatenate(
        [x, y, z, d, rsq, rq, mind,
         row(gtx), row(gty), row(gtz), row(gsx), row(gsy), row(gsz),
         jnp.zeros((3, BE), jnp.float32)], axis=0)
    h = jnp.tanh(
        jnp.dot(w1t[...], feats, preferred_element_type=jnp.float32) + b1c[...])
    v = jnp.dot(w2t[...], h, preferred_element_type=jnp.float32) + b2c[...]
    ox[...] = v[0:1, :].reshape(BE)
    oy[...] = v[1:2, :].reshape(BE)
    oz[...] = v[2:3, :].reshape(BE)


def _tc_mlp(comps, w1t, b1c, w2t, b2c, prm):
    grid = (N_EDGES // BE,)
    espec = pl.BlockSpec((BE,), lambda i: (i,))
    wspec = lambda shape: pl.BlockSpec(shape, lambda i: tuple(0 for _ in shape))
    e = jax.ShapeDtypeStruct((N_EDGES,), jnp.float32)
    return pl.pallas_call(
        _mlp_body,
        grid=grid,
        in_specs=[espec] * 9 + [wspec((32, 16)), wspec((32, 1)),
                                wspec((3, 32)), wspec((3, 1)), wspec((1, 2))],
        out_specs=[espec] * 3,
        out_shape=(e, e, e),
    )(*comps, w1t, b1c, w2t, b2c, prm)


def _scatter_body(ti_hbm, vx_hbm, vy_hbm, vz_hbm, out_hbm,
                  ti_v, vv_v, zb_v, shx, shy, shz):
    cid = lax.axis_index("c")
    sid = lax.axis_index("s")
    wid = sid * NC + cid
    shs = (shx, shy, shz)
    vins = (vx_hbm, vy_hbm, vz_hbm)

    def zb(i, _):
        zb_v[pl.ds(i * LANES, LANES)] = jnp.zeros((LANES,), jnp.float32)
        return 0

    lax.fori_loop(0, NPT // LANES, zb, 0)
    for sh in shs:
        pltpu.sync_copy(zb_v, sh.at[pl.ds(sid * NPT, NPT)])
    plsc.subcore_barrier()

    def cbase(c):
        return wid * PW + jnp.minimum(c, NCHUNK - 1) * CB

    def issue_in(c, b, semi):
        base = cbase(c)
        pltpu.async_copy(ti_hbm.at[pl.ds(base, CB)], ti_v.at[b], semi)
        for comp in range(3):
            pltpu.async_copy(vins[comp].at[pl.ds(base, CB)],
                             vv_v.at[b, comp], semi)

    def drain_in(b, semi):
        pltpu.make_async_copy(ti_hbm.at[pl.ds(0, CB)], ti_v.at[b], semi).wait()
        for comp in range(3):
            pltpu.make_async_copy(vins[comp].at[pl.ds(0, CB)],
                                  vv_v.at[b, comp], semi).wait()

    def issue_add(b, sema):
        for comp in range(3):
            pltpu.async_copy(vv_v.at[b, comp], shs[comp].at[ti_v.at[b]],
                             sema, add=True)

    def drain_add(b, sema):
        for comp in range(3):
            pltpu.make_async_copy(vv_v.at[b, comp],
                                  shs[comp].at[ti_v.at[b]], sema).wait()

    def run_chunk(c, b, first, semi, sema):
        drain_in(b, semi)
        if not first:
            drain_add(1 - b, sema)
        issue_in(c + 1, 1 - b, semi)
        issue_add(b, sema)

    def scatter_phase(semi, sema):
        issue_in(0, 0, semi)
        run_chunk(0, 0, True, semi, sema)
        run_chunk(1, 1, False, semi, sema)

        def pair(j, _):
            run_chunk(2 * j, 0, False, semi, sema)
            run_chunk(2 * j + 1, 1, False, semi, sema)
            return 0

        lax.fori_loop(1, NCHUNK // 2, pair, 0)
        drain_add(1, sema)
        drain_in(0, semi)

    pl.run_scoped(scatter_phase,
                  pltpu.SemaphoreType.DMA, pltpu.SemaphoreType.DMA)
    plsc.subcore_barrier()
    for comp, sh in enumerate(shs):
        pltpu.sync_copy(sh.at[pl.ds(sid * NPT, NPT)], zb_v)
        pltpu.sync_copy(
            zb_v, out_hbm.at[pl.ds(cid * 3 * NP + comp * NP + sid * NPT, NPT)])


def _sc_scatter(ti, vx, vy, vz):
    mesh = plsc.VectorSubcoreMesh(core_axis_name="c", subcore_axis_name="s")
    return pl.kernel(
        _scatter_body,
        out_type=jax.ShapeDtypeStruct((NC * 3 * NP,), jnp.float32),
        mesh=mesh,
        scratch_types=[
            pltpu.VMEM((2, CB), jnp.int32),
            pltpu.VMEM((2, 3, CB), jnp.float32),
            pltpu.VMEM((NPT,), jnp.float32),
            pltpu.VMEM_SHARED((NP,), jnp.float32),
            pltpu.VMEM_SHARED((NP,), jnp.float32),
            pltpu.VMEM_SHARED((NP,), jnp.float32),
        ],
        compiler_params=pltpu.CompilerParams(use_tc_tiling_on_sc=False,
                                             needs_layout_passes=False),
    )(ti, vx, vy, vz)


def kernel(rel_selected, target_indices, source_indices, force, viscosity,
           W1, b1, W2, b2, median, contact_distance):
    ti = target_indices.astype(jnp.int32)
    si = source_indices.astype(jnp.int32)
    n = force.shape[0]
    fp = jnp.concatenate([force, jnp.zeros((n, DPAD - 3), jnp.float32)], axis=1)
    relx = rel_selected[:, 0]
    rely = rel_selected[:, 1]
    relz = rel_selected[:, 2]
    w1t = jnp.concatenate([W1[:13], jnp.zeros((3, W1.shape[1]), jnp.float32)],
                          axis=0).T
    b1c = (b1 + viscosity * W1[13])[:, None]
    w2t = W2.T
    b2c = b2[:, None]
    prm = jnp.stack([median, contact_distance]).reshape(1, 2)

    gt = _sc_gather(fp, ti, si)
    vx, vy, vz = _tc_mlp((relx, rely, relz) + tuple(gt), w1t, b1c, w2t, b2c, prm)
    parts = _sc_scatter(ti, vx, vy, vz).reshape(NC, 3, NP)
    return (parts[0] + parts[1])[:, :n].T
